# baseline (speedup 1.0000x reference)
import functools

import jax
import jax.numpy as jnp
from jax import lax
from jax.experimental import pallas as pl
from jax.experimental.pallas import tpu as pltpu
from jax.experimental.pallas import tpu_sc as plsc

EMBED = 64
_B = 1024
_L = 200

_info = plsc.get_sparse_core_info()
_NC, _NS = _info.num_cores, _info.num_subcores
_NW = _NC * _NS
_BLK = 128
_JB = _B // _BLK
_NBLK = _L * _JB
_BPW = _NBLK // _NW

_mesh = plsc.VectorSubcoreMesh(core_axis_name="c", subcore_axis_name="s")


@functools.partial(
    pl.kernel,
    mesh=_mesh,
    out_type=jax.ShapeDtypeStruct((_L, 8, _JB, 8, _BLK), jnp.float32),
    compiler_params=pltpu.CompilerParams(
        use_tc_tiling_on_sc=False, needs_layout_passes=False),
    scratch_types=[
        pltpu.VMEM((_BLK,), jnp.int32),
        pltpu.VMEM((_BLK,), jnp.int32),
        pltpu.VMEM((_BLK, 128), jnp.float32),
        pltpu.VMEM((8, 8, _BLK), jnp.float32),
        pltpu.SemaphoreType.DMA,
    ],
)
def _gather(idx_hbm, table2_hbm, out_hbm, idx_v, pidx_v, pair_v, out_v, sem):
    wid = lax.axis_index("s") * _NC + lax.axis_index("c")

    def body(i, carry):
        blk = wid * _BPW + i
        l = blk // _JB
        j = blk % _JB
        pltpu.async_copy(idx_hbm.at[l, pl.ds(j * _BLK, _BLK)], idx_v, sem).wait()
        for g in range(_BLK // 16):
            pidx_v[pl.ds(g * 16, 16)] = idx_v[pl.ds(g * 16, 16)] >> 1
        pltpu.async_copy(table2_hbm.at[pidx_v], pair_v, sem).wait()
        for g in range(_BLK // 16):
            b_ids = lax.iota(jnp.int32, 16) + g * 16
            h64 = (idx_v[pl.ds(g * 16, 16)] & 1) * 64
            for e in range(EMBED):
                out_v[e // 8, e % 8, pl.ds(g * 16, 16)] = plsc.load_gather(
                    pair_v, [b_ids, h64 + e])
        pltpu.async_copy(out_v, out_hbm.at[l, :, j, :, :], sem).wait()
        return carry

    lax.fori_loop(0, _BPW, body, 0)


def kernel(inputtokens, table):
    idxT = jnp.transpose(inputtokens).astype(jnp.int32)
    table2 = table.reshape(500000, 128)
    out5 = _gather(idxT, table2)
    out = jnp.transpose(out5, (2, 4, 0, 1, 3))
    return out.reshape(_B, _L, EMBED)


# speedup vs baseline: 1.0794x; 1.0794x over previous
import functools

import jax
import jax.numpy as jnp
from jax import lax
from jax.experimental import pallas as pl
from jax.experimental.pallas import tpu as pltpu
from jax.experimental.pallas import tpu_sc as plsc

EMBED = 64
_B = 1024
_L = 200

_info = plsc.get_sparse_core_info()
_NC, _NS = _info.num_cores, _info.num_subcores
_NW = _NC * _NS
_BLK = 128
_JB = _B // _BLK
_NBLK = _L * _JB
_BPW = _NBLK // _NW

_mesh = plsc.VectorSubcoreMesh(core_axis_name="c", subcore_axis_name="s")


@functools.partial(
    pl.kernel,
    mesh=_mesh,
    out_type=jax.ShapeDtypeStruct((_L, 8, _JB, 8, _BLK), jnp.float32),
    compiler_params=pltpu.CompilerParams(
        use_tc_tiling_on_sc=False, needs_layout_passes=False),
    scratch_types=[
        pltpu.VMEM((2, _BLK), jnp.int32),
        pltpu.VMEM((2, _BLK), jnp.int32),
        pltpu.VMEM((2, _BLK, 128), jnp.float32),
        pltpu.VMEM((2, 8, 8, _BLK), jnp.float32),
        pltpu.SemaphoreType.DMA,
        pltpu.SemaphoreType.DMA,
        pltpu.SemaphoreType.DMA,
        pltpu.SemaphoreType.DMA,
        pltpu.SemaphoreType.DMA,
        pltpu.SemaphoreType.DMA,
    ],
)
def _gather(idx_hbm, table2_hbm, out_hbm, idx_vv, pidx_vv, pair_vv, out_vv,
            is0, is1, gs0, gs1, os0, os1):
    wid = lax.axis_index("s") * _NC + lax.axis_index("c")
    isem = (is0, is1)
    gsem = (gs0, gs1)
    osem = (os0, os1)

    def lj(i):
        blk = wid * _BPW + i
        return blk // _JB, blk % _JB

    def fire_front(i, p):
        l, jb = lj(i)
        idx_p = idx_vv.at[p]
        pidx_p = pidx_vv.at[p]
        pltpu.async_copy(idx_hbm.at[l, pl.ds(jb * _BLK, _BLK)], idx_p,
                         isem[p]).wait()
        for g in range(_BLK // 16):
            pidx_p[pl.ds(g * 16, 16)] = idx_p[pl.ds(g * 16, 16)] >> 1
        pltpu.async_copy(table2_hbm.at[pidx_p], pair_vv.at[p], gsem[p])

    def drain_gather(p):
        pltpu.make_async_copy(table2_hbm.at[pl.ds(0, _BLK)], pair_vv.at[p],
                              gsem[p]).wait()

    def drain_out(p):
        pltpu.make_async_copy(out_vv.at[p], out_hbm.at[0, :, 0, :, :],
                              osem[p]).wait()

    def back(i, p):
        l, jb = lj(i)
        idx_p = idx_vv.at[p]
        pair_p = pair_vv.at[p]
        out_p = out_vv.at[p]
        h64s = [(idx_p[pl.ds(g * 16, 16)] & 1) * 64
                for g in range(_BLK // 16)]
        b_ids = [lax.iota(jnp.int32, 16) + g * 16 for g in range(_BLK // 16)]

        def er_body(er, carry):
            for es in range(8):
                e = er * 8 + es
                for g in range(_BLK // 16):
                    out_p[er, es, pl.ds(g * 16, 16)] = plsc.load_gather(
                        pair_p, [b_ids[g], h64s[g] + e])
            return carry

        lax.fori_loop(0, 8, er_body, 0)
        pltpu.async_copy(out_p, out_hbm.at[l, :, jb, :, :], osem[p])

    fire_front(0, 0)
    fire_front(1, 1)
    drain_gather(0)
    back(0, 0)
    fire_front(2, 0)
    drain_gather(1)
    back(1, 1)
    fire_front(3, 1)

    def body(j, carry):
        i0 = 2 * j
        drain_gather(0)
        drain_out(0)
        back(i0, 0)
        fire_front(i0 + 2, 0)
        drain_gather(1)
        drain_out(1)
        back(i0 + 1, 1)
        fire_front(i0 + 3, 1)
        return carry

    lax.fori_loop(1, _BPW // 2 - 1, body, 0)

    drain_gather(0)
    drain_out(0)
    back(_BPW - 2, 0)
    drain_gather(1)
    drain_out(1)
    back(_BPW - 1, 1)
    drain_out(0)
    drain_out(1)


def kernel(inputtokens, table):
    idxT = jnp.transpose(inputtokens).astype(jnp.int32)
    table2 = table.reshape(500000, 128)
    out5 = _gather(idxT, table2)
    out = jnp.transpose(out5, (2, 4, 0, 1, 3))
    return out.reshape(_B, _L, EMBED)
